# baseline (speedup 1.0000x reference)
import functools

import jax
import jax.numpy as jnp
from jax import lax
from jax.experimental import pallas as pl
from jax.experimental.pallas import tpu as pltpu
from jax.experimental.pallas import tpu_sc as plsc

CHUNK = 128


def _make_gather(n_total, d):
    info = plsc.get_sparse_core_info()
    nc, ns = info.num_cores, info.num_subcores
    nw = nc * ns
    per_w = n_total // nw
    n_chunks = per_w // CHUNK

    mesh = plsc.VectorSubcoreMesh(core_axis_name="c", subcore_axis_name="s")

    @functools.partial(
        pl.kernel,
        mesh=mesh,
        out_type=jax.ShapeDtypeStruct((n_total, d), jnp.float32),
        scratch_types=[
            pltpu.VMEM((n_chunks, CHUNK), jnp.int32),
            pltpu.VMEM((CHUNK, d), jnp.float32),
            pltpu.SemaphoreType.DMA,
        ],
        compiler_params=pltpu.CompilerParams(use_tc_tiling_on_sc=False),
    )
    def gather_kernel(idx_hbm, table_hbm, out_hbm, idx_v, rows_v, sem):
        wid = lax.axis_index("s") * nc + lax.axis_index("c")
        pltpu.sync_copy(idx_hbm.at[wid], idx_v)
        base = wid * per_w

        def body(j, carry):
            pltpu.async_copy(table_hbm.at[idx_v.at[j]], rows_v, sem).wait()
            pltpu.sync_copy(rows_v, out_hbm.at[pl.ds(base + j * CHUNK, CHUNK)])
            return carry

        lax.fori_loop(0, n_chunks, body, 0)

    return gather_kernel, nw, n_chunks


def kernel(input_tensor, weight):
    b, l = input_tensor.shape
    _, d = weight.shape
    n_total = b * l
    gather_kernel, nw, n_chunks = _make_gather(n_total, d)
    idx = input_tensor.reshape(nw, n_chunks, CHUNK).astype(jnp.int32)
    out = gather_kernel(idx, weight)
    return out.reshape(b, l, d)


# speedup vs baseline: 1.1102x; 1.1102x over previous
import functools

import jax
import jax.numpy as jnp
from jax import lax
from jax.experimental import pallas as pl
from jax.experimental.pallas import tpu as pltpu
from jax.experimental.pallas import tpu_sc as plsc

CHUNK = 128
NB = 4


def _make_gather(n_total, d):
    info = plsc.get_sparse_core_info()
    nc, ns = info.num_cores, info.num_subcores
    nw = nc * ns
    per_w = n_total // nw
    n_chunks = per_w // CHUNK
    n_groups = n_chunks // NB
    assert n_chunks == n_groups * NB and n_groups % 2 == 0

    mesh = plsc.VectorSubcoreMesh(core_axis_name="c", subcore_axis_name="s")

    @functools.partial(
        pl.kernel,
        mesh=mesh,
        out_type=jax.ShapeDtypeStruct((n_total, d), jnp.float32),
        scratch_types=[
            pltpu.VMEM((n_chunks, CHUNK), jnp.int32),
            pltpu.VMEM((2, NB, CHUNK, d), jnp.float32),
            pltpu.SemaphoreType.DMA,
            pltpu.SemaphoreType.DMA,
            pltpu.SemaphoreType.DMA,
            pltpu.SemaphoreType.DMA,
        ],
        compiler_params=pltpu.CompilerParams(use_tc_tiling_on_sc=False),
    )
    def gather_kernel(idx_hbm, table_hbm, out_hbm, idx_v, bufs, g0, g1, s0, s1):
        wid = lax.axis_index("s") * nc + lax.axis_index("c")
        pltpu.sync_copy(idx_hbm.at[wid], idx_v)
        base = wid * per_w
        gsem = (g0, g1)
        ssem = (s0, s1)

        def issue_gathers(g, h):
            for b in range(NB):
                pltpu.async_copy(
                    table_hbm.at[idx_v.at[g * NB + b]], bufs.at[h, b], gsem[h])

        def drain_gathers(h):
            for b in range(NB):
                pltpu.make_async_copy(
                    table_hbm.at[idx_v.at[b]], bufs.at[h, b], gsem[h]).wait()

        def issue_stores(g, h):
            for b in range(NB):
                j = g * NB + b
                pltpu.async_copy(
                    bufs.at[h, b], out_hbm.at[pl.ds(base + j * CHUNK, CHUNK)],
                    ssem[h])

        def drain_stores(h):
            for b in range(NB):
                pltpu.make_async_copy(
                    bufs.at[h, b], out_hbm.at[pl.ds(base, CHUNK)], ssem[h]).wait()

        issue_gathers(0, 0)

        def body(p, carry):
            gA = 2 * p
            gB = gA + 1
            issue_gathers(gB, 1)
            drain_gathers(0)
            issue_stores(gA, 0)
            drain_gathers(1)
            issue_stores(gB, 1)
            drain_stores(0)

            @pl.when(gA + 2 < n_groups)
            def _():
                issue_gathers(gA + 2, 0)

            drain_stores(1)
            return carry

        lax.fori_loop(0, n_groups // 2, body, 0)

    return gather_kernel, nw, n_chunks


def kernel(input_tensor, weight):
    b, l = input_tensor.shape
    _, d = weight.shape
    n_total = b * l
    gather_kernel, nw, n_chunks = _make_gather(n_total, d)
    idx = input_tensor.reshape(nw, n_chunks, CHUNK).astype(jnp.int32)
    out = gather_kernel(idx, weight)
    return out.reshape(b, l, d)
